# Initial kernel scaffold; baseline (speedup 1.0000x reference)
import jax
import jax.numpy as jnp
from jax.experimental import pallas as pl


def kernel(distance_matrix, groups, group_weights):
    raise NotImplementedError("write your pallas kernel here")


# speedup vs baseline: 648.2547x; 648.2547x over previous
import functools
import jax
import jax.numpy as jnp
from jax import lax
from jax.experimental import pallas as pl
from jax.experimental.pallas import tpu as pltpu
from jax.experimental.pallas import tpu_sc as plsc

R, C = 16384, 1024
TOT = R * C
NW = 32
PER_W = TOT // NW
CHUNK = 16384
NCH = PER_W // CHUNK
NBUF = 2
NOUTER = NCH // NBUF
TBL = 128
L = 16


def _body(dist_hbm, grp_hbm, tbl_hbm, out_hbm,
          tbl_v, dist_v0, dist_v1, grp_v0, grp_v1, out_v0, out_v1,
          sd0, sd1, sg0, sg1, so0, so1):
  dist_bufs = [dist_v0, dist_v1]
  grp_bufs = [grp_v0, grp_v1]
  out_bufs = [out_v0, out_v1]
  cid = lax.axis_index("c")
  sid = lax.axis_index("s")
  wid = sid * 2 + cid
  base = wid * PER_W

  in_sems_d = [sd0, sd1]
  in_sems_g = [sg0, sg1]
  out_sems = [so0, so1]

  pltpu.sync_copy(tbl_hbm, tbl_v)

  def in_slices(cg):
    off = base + cg * CHUNK
    return dist_hbm.at[pl.ds(off, CHUNK)], grp_hbm.at[pl.ds(off, CHUNK)]

  for b in range(NBUF):
    dsl, gsl = in_slices(b)
    pltpu.async_copy(dsl, dist_bufs[b], in_sems_d[b])
    pltpu.async_copy(gsl, grp_bufs[b], in_sems_g[b])

  @pl.loop(0, NOUTER)
  def outer(j):
    for b in range(NBUF):
      cg = j * NBUF + b
      dsl, gsl = in_slices(cg)
      pltpu.make_async_copy(dsl, dist_bufs[b], in_sems_d[b]).wait()
      pltpu.make_async_copy(gsl, grp_bufs[b], in_sems_g[b]).wait()

      @pl.when(j > 0)
      def _():
        prev = (j - 1) * NBUF + b
        pltpu.make_async_copy(
            out_bufs[b], out_hbm.at[pl.ds(base + prev * CHUNK, CHUNK)],
            out_sems[b]).wait()

      db = dist_bufs[b]
      gb = grp_bufs[b]
      ob = out_bufs[b]

      @plsc.parallel_loop(0, CHUNK, L, unroll=8)
      def inner(i):
        g = gb[pl.ds(i, L)]
        w = plsc.load_gather(tbl_v, [g])
        d = db[pl.ds(i, L)]
        ob[pl.ds(i, L)] = jnp.maximum(d * w, 0.0)

      pltpu.async_copy(ob, out_hbm.at[pl.ds(base + cg * CHUNK, CHUNK)],
                       out_sems[b])

      @pl.when(j < NOUTER - 1)
      def _():
        nxt = cg + NBUF
        dsl2, gsl2 = in_slices(nxt)
        pltpu.async_copy(dsl2, dist_bufs[b], in_sems_d[b])
        pltpu.async_copy(gsl2, grp_bufs[b], in_sems_g[b])

  for b in range(NBUF):
    cg = (NOUTER - 1) * NBUF + b
    pltpu.make_async_copy(
        out_bufs[b], out_hbm.at[pl.ds(base + cg * CHUNK, CHUNK)],
        out_sems[b]).wait()


@jax.jit
def _run(dist_flat, grp_flat, tbl):
  mesh = plsc.VectorSubcoreMesh(core_axis_name="c", subcore_axis_name="s")
  return pl.kernel(
      _body,
      out_type=jax.ShapeDtypeStruct((TOT,), jnp.float32),
      mesh=mesh,
      compiler_params=pltpu.CompilerParams(needs_layout_passes=False),
      scratch_types=[
          pltpu.VMEM((TBL,), jnp.float32),
          pltpu.VMEM((CHUNK,), jnp.float32),
          pltpu.VMEM((CHUNK,), jnp.float32),
          pltpu.VMEM((CHUNK,), jnp.int32),
          pltpu.VMEM((CHUNK,), jnp.int32),
          pltpu.VMEM((CHUNK,), jnp.float32),
          pltpu.VMEM((CHUNK,), jnp.float32),
          pltpu.SemaphoreType.DMA,
          pltpu.SemaphoreType.DMA,
          pltpu.SemaphoreType.DMA,
          pltpu.SemaphoreType.DMA,
          pltpu.SemaphoreType.DMA,
          pltpu.SemaphoreType.DMA,
      ],
  )(dist_flat, grp_flat, tbl)


def kernel(distance_matrix, groups, group_weights):
  tbl = jnp.zeros((TBL,), jnp.float32).at[:group_weights.shape[0]].set(
      group_weights[:, 0])
  out = _run(distance_matrix.reshape(-1),
             groups.reshape(-1).astype(jnp.int32), tbl)
  return out.reshape(R, C)


# speedup vs baseline: 1819.8900x; 2.8074x over previous
import jax
import jax.numpy as jnp
from jax import lax
from jax.experimental import pallas as pl
from jax.experimental.pallas import tpu as pltpu
from jax.experimental.pallas import tpu_sc as plsc

R, C = 16384, 1024
NW = 32
ROWS_W = R // NW
CR = 16
NCH = ROWS_W // CR
NBUF = 2
NOUTER = NCH // NBUF
TBL = 128
L = 16
VPC = CR * C // L


def _body(dist_hbm, grp_hbm, tbl_hbm, out_hbm,
          tbl_v, dist_v0, dist_v1, grp_v0, grp_v1, out_v0, out_v1,
          sd0, sd1, sg0, sg1, so0, so1):
  dist_bufs = [dist_v0, dist_v1]
  grp_bufs = [grp_v0, grp_v1]
  out_bufs = [out_v0, out_v1]
  cid = lax.axis_index("c")
  sid = lax.axis_index("s")
  wid = sid * 2 + cid
  base = wid * ROWS_W

  in_sems_d = [sd0, sd1]
  in_sems_g = [sg0, sg1]
  out_sems = [so0, so1]

  pltpu.sync_copy(tbl_hbm, tbl_v)

  lanes = lax.iota(jnp.int32, L)

  def in_slices(cg):
    row = base + cg * CR
    return dist_hbm.at[pl.ds(row, CR)], grp_hbm.at[pl.ds(row, CR)]

  def out_slice(cg):
    return out_hbm.at[pl.ds(base + cg * CR, CR)]

  for b in range(NBUF):
    dsl, gsl = in_slices(b)
    pltpu.async_copy(dsl, dist_bufs[b], in_sems_d[b])
    pltpu.async_copy(gsl, grp_bufs[b], in_sems_g[b])

  @pl.loop(0, NOUTER)
  def outer(j):
    for b in range(NBUF):
      cg = j * NBUF + b
      dsl, gsl = in_slices(cg)
      pltpu.make_async_copy(dsl, dist_bufs[b], in_sems_d[b]).wait()
      pltpu.make_async_copy(gsl, grp_bufs[b], in_sems_g[b]).wait()

      @pl.when(j > 0)
      def _():
        pltpu.make_async_copy(out_bufs[b], out_slice((j - 1) * NBUF + b),
                              out_sems[b]).wait()

      db = dist_bufs[b]
      gb = grp_bufs[b]
      ob = out_bufs[b]

      @plsc.parallel_loop(0, VPC, 1, unroll=8)
      def inner(i):
        r = jnp.full((L,), i >> 6, jnp.int32)
        c = lanes + ((i & 63) << 4)
        g = plsc.load_gather(gb, [r, c])
        w = plsc.load_gather(tbl_v, [g])
        d = plsc.load_gather(db, [r, c])
        plsc.store_scatter(ob, [r, c], jnp.maximum(d * w, 0.0))

      pltpu.async_copy(ob, out_slice(cg), out_sems[b])

      @pl.when(j < NOUTER - 1)
      def _():
        dsl2, gsl2 = in_slices(cg + NBUF)
        pltpu.async_copy(dsl2, dist_bufs[b], in_sems_d[b])
        pltpu.async_copy(gsl2, grp_bufs[b], in_sems_g[b])

  for b in range(NBUF):
    pltpu.make_async_copy(out_bufs[b], out_slice((NOUTER - 1) * NBUF + b),
                          out_sems[b]).wait()


@jax.jit
def _run(dist, grp, tbl):
  mesh = plsc.VectorSubcoreMesh(core_axis_name="c", subcore_axis_name="s")
  return pl.kernel(
      _body,
      out_type=jax.ShapeDtypeStruct((R, C), jnp.float32),
      mesh=mesh,
      compiler_params=pltpu.CompilerParams(needs_layout_passes=False),
      scratch_types=[
          pltpu.VMEM((TBL,), jnp.float32),
          pltpu.VMEM((CR, C), jnp.float32),
          pltpu.VMEM((CR, C), jnp.float32),
          pltpu.VMEM((CR, C), jnp.int32),
          pltpu.VMEM((CR, C), jnp.int32),
          pltpu.VMEM((CR, C), jnp.float32),
          pltpu.VMEM((CR, C), jnp.float32),
          pltpu.SemaphoreType.DMA,
          pltpu.SemaphoreType.DMA,
          pltpu.SemaphoreType.DMA,
          pltpu.SemaphoreType.DMA,
          pltpu.SemaphoreType.DMA,
          pltpu.SemaphoreType.DMA,
      ],
  )(dist, grp, tbl)


def kernel(distance_matrix, groups, group_weights):
  tbl = jnp.zeros((TBL,), jnp.float32).at[:group_weights.shape[0]].set(
      group_weights[:, 0])
  return _run(distance_matrix, groups.astype(jnp.int32), tbl)
